# baseline (speedup 1.0000x reference)
import functools

import jax
import jax.numpy as jnp
from jax import lax
from jax.experimental import pallas as pl
from jax.experimental.pallas import tpu as pltpu
from jax.experimental.pallas import tpu_sc as plsc

VOC = 100000
DIM = 64
SEQ = 50
BN = 4096
NC = 2
NS = 16
NW = NC * NS
D_PER_W = DIM // NW
PAIRS = SEQ // 2

_mesh = plsc.VectorSubcoreMesh(core_axis_name="c", subcore_axis_name="s")


@functools.partial(
    pl.kernel,
    mesh=_mesh,
    out_type=jax.ShapeDtypeStruct((SEQ, DIM, BN), jnp.float32),
    compiler_params=pltpu.CompilerParams(
        needs_layout_passes=False,
        disable_bounds_checks=True,
        disable_semaphore_checks=True,
    ),
    scratch_types=[
        pltpu.VMEM((VOC,), jnp.float32),
        [pltpu.VMEM((BN,), jnp.int32) for _ in range(2)],
        [pltpu.VMEM((BN,), jnp.float32) for _ in range(2)],
        pltpu.VMEM_SHARED((SEQ * BN,), jnp.int32),
        pltpu.SemaphoreType.DMA,
        [pltpu.SemaphoreType.DMA for _ in range(2)],
        [pltpu.SemaphoreType.DMA for _ in range(2)],
    ],
)
def _embed(xt_hbm, tablet_hbm, out_hbm, trow, xrows, orows, x_sp, tsem, xsems, wsems):
    sid = lax.axis_index("s")
    wid = sid * NC + lax.axis_index("c")

    tcp0 = pltpu.async_copy(tablet_hbm.at[wid * D_PER_W], trow, tsem)

    for k in range((SEQ + NS - 1) // NS):
        t_load = k * NS + sid

        @pl.when(t_load < SEQ)
        def _load_x():
            pltpu.sync_copy(xt_hbm.at[t_load], x_sp.at[pl.ds(t_load * BN, BN)])

    plsc.subcore_barrier()

    def _drain_x(b):
        pltpu.make_async_copy(xt_hbm.at[0], xrows[b], xsems[b]).wait()

    def _drain_w(b):
        pltpu.make_async_copy(orows[b], out_hbm.at[0, 0], wsems[b]).wait()

    for dd in range(D_PER_W):
        d = wid * D_PER_W + dd
        tcp = tcp0 if dd == 0 else pltpu.async_copy(tablet_hbm.at[d], trow, tsem)
        pltpu.async_copy(x_sp.at[pl.ds(0, BN)], xrows[0], xsems[0])
        pltpu.async_copy(x_sp.at[pl.ds(BN, BN)], xrows[1], xsems[1])
        tcp.wait()
        first_phase = dd == 0

        def _step(i, b, drain_w, prefetch):
            t = 2 * i + b
            _drain_x(b)
            if drain_w:
                _drain_w(b)
            xrow = xrows[b]
            orow = orows[b]

            @plsc.parallel_loop(0, BN, 16, unroll=16)
            def _gather(j):
                idx = xrow[pl.ds(j, 16)]
                orow[pl.ds(j, 16)] = plsc.load_gather(trow, [idx])

            pltpu.async_copy(orow, out_hbm.at[t, d], wsems[b])
            if prefetch:
                pltpu.async_copy(
                    x_sp.at[pl.ds((t + 2) * BN, BN)], xrows[b], xsems[b]
                )

        def _pair(i, carry):
            for b in range(2):
                _step(i, b, drain_w=True, prefetch=True)
            return carry

        for b in range(2):
            _step(0, b, drain_w=not first_phase, prefetch=True)
        lax.fori_loop(1, PAIRS - 1, _pair, 0)
        for b in range(2):
            _step(PAIRS - 1, b, drain_w=True, prefetch=False)
    _drain_w(0)
    _drain_w(1)


def kernel(x, table):
    out_t = _embed(x.T, table.T)
    return jnp.transpose(out_t, (2, 0, 1))


# speedup vs baseline: 1.0188x; 1.0188x over previous
import functools

import jax
import jax.numpy as jnp
from jax import lax
from jax.experimental import pallas as pl
from jax.experimental.pallas import tpu as pltpu
from jax.experimental.pallas import tpu_sc as plsc

VOC = 100000
DIM = 64
SEQ = 50
BN = 4096
NC = 2
NS = 16
NW = NC * NS
D_PER_W = DIM // NW
PAIRS = SEQ // 2

_mesh = plsc.VectorSubcoreMesh(core_axis_name="c", subcore_axis_name="s")


@functools.partial(
    pl.kernel,
    mesh=_mesh,
    out_type=jax.ShapeDtypeStruct((SEQ, DIM, BN), jnp.float32),
    compiler_params=pltpu.CompilerParams(
        needs_layout_passes=False,
        disable_bounds_checks=True,
        disable_semaphore_checks=True,
    ),
    scratch_types=[
        pltpu.VMEM((VOC,), jnp.float32),
        [pltpu.VMEM((BN,), jnp.int32) for _ in range(2)],
        [pltpu.VMEM((BN,), jnp.float32) for _ in range(2)],
        pltpu.VMEM_SHARED((SEQ * BN,), jnp.int32),
        pltpu.SemaphoreType.DMA,
        [pltpu.SemaphoreType.DMA for _ in range(2)],
        [pltpu.SemaphoreType.DMA for _ in range(2)],
    ],
)
def _embed(xt_hbm, tablet_hbm, out_hbm, trow, xrows, orows, x_sp, tsem, xsems, wsems):
    sid = lax.axis_index("s")
    wid = sid * NC + lax.axis_index("c")

    tcp0 = pltpu.async_copy(tablet_hbm.at[wid * D_PER_W], trow, tsem)

    for k in range((SEQ + NS - 1) // NS):
        t_load = k * NS + sid

        @pl.when(t_load < SEQ)
        def _load_x():
            pltpu.sync_copy(xt_hbm.at[t_load], x_sp.at[pl.ds(t_load * BN, BN)])

    plsc.subcore_barrier()

    def _drain_x(b):
        pltpu.make_async_copy(xt_hbm.at[0], xrows[b], xsems[b]).wait()

    def _drain_w(b):
        pltpu.make_async_copy(orows[b], out_hbm.at[0, 0], wsems[b]).wait()

    for dd in range(D_PER_W):
        d = wid * D_PER_W + dd
        tcp = tcp0 if dd == 0 else pltpu.async_copy(tablet_hbm.at[d], trow, tsem)
        pltpu.async_copy(x_sp.at[pl.ds(0, BN)], xrows[0], xsems[0])
        pltpu.async_copy(x_sp.at[pl.ds(BN, BN)], xrows[1], xsems[1])
        tcp.wait()
        first_phase = dd == 0

        def _pair(i, carry):
            for b in range(2):
                t = 2 * i + b
                _drain_x(b)
                if first_phase:

                    @pl.when(i > 0)
                    def _wait_prev_write():
                        _drain_w(b)

                else:
                    _drain_w(b)
                xrow = xrows[b]
                orow = orows[b]

                @plsc.parallel_loop(0, BN, 16, unroll=16)
                def _gather(j):
                    idx = xrow[pl.ds(j, 16)]
                    orow[pl.ds(j, 16)] = plsc.load_gather(trow, [idx])

                pltpu.async_copy(orow, out_hbm.at[t, d], wsems[b])

                @pl.when(i + 1 < PAIRS)
                def _prefetch_x():
                    pltpu.async_copy(
                        x_sp.at[pl.ds((t + 2) * BN, BN)], xrows[b], xsems[b]
                    )

            return carry

        lax.fori_loop(0, PAIRS, _pair, 0)
    _drain_w(0)
    _drain_w(1)


def kernel(x, table):
    out_t = _embed(x.T, table.T)
    return jnp.transpose(out_t, (2, 0, 1))
